# baseline (speedup 1.0000x reference)
import jax
import jax.numpy as jnp
from jax import lax
from jax.experimental import pallas as pl
from jax.experimental.pallas import tpu as pltpu
from jax.experimental.pallas import tpu_sc as plsc

N = 10000
E = 320000
N_PAD = 10240
C = 80
ROWS_E = E // C
N_CORES = 2
N_SUB = 16
TILES = N_CORES * N_SUB
CPT = ROWS_E // TILES
RPT = N_PAD // N_SUB


NBUF = 5
PREF = 4


def _sc_agg(dout, with_cnt):
  nblk = dout // 16
  mesh = plsc.VectorSubcoreMesh(core_axis_name="c", subcore_axis_name="s")
  out_type = [jax.ShapeDtypeStruct((N_PAD, 128), jnp.float32)]
  scratch = [
      pltpu.VMEM((CPT, C), jnp.int32),
      pltpu.VMEM((CPT, C), jnp.int32),
      pltpu.VMEM((CPT, C), jnp.float32),
      pltpu.VMEM((NBUF, C, dout), jnp.float32),
      pltpu.VMEM((C, dout), jnp.float32),
      pltpu.VMEM_SHARED((N_PAD, dout), jnp.float32),
  ] + [pltpu.SemaphoreType.DMA] * (2 * NBUF)
  if with_cnt:
    out_type.append(jax.ShapeDtypeStruct((N_PAD, 128), jnp.float32))
    scratch += [
        pltpu.VMEM((C, 16), jnp.float32),
        pltpu.VMEM((C, 16), jnp.float32),
        pltpu.VMEM_SHARED((N_PAD, 16), jnp.float32),
        pltpu.SemaphoreType.DMA,
    ]

  def body(hl, srcr, dstr, wr, *rest):
    if with_cnt:
      agg_o, cnt_o = rest[0], rest[1]
      (src_v, dst_v, w_v, rows_v, z_v, acc) = rest[2:8]
      gsems = rest[8:8 + NBUF]
      ssems = rest[8 + NBUF:8 + 2 * NBUF]
      e0_v, z16_v, cacc, csem = rest[8 + 2 * NBUF:]
    else:
      agg_o = rest[0]
      (src_v, dst_v, w_v, rows_v, z_v, acc) = rest[1:7]
      gsems = rest[7:7 + NBUF]
      ssems = rest[7 + NBUF:7 + 2 * NBUF]
    c = lax.axis_index("c")
    s = lax.axis_index("s")
    tile = c * N_SUB + s

    zvec = jnp.zeros((16,), jnp.float32)
    onevec = jnp.ones((16,), jnp.float32)

    def zfill(i, carry):
      for k in range(nblk):
        z_v[i, pl.ds(k * 16, 16)] = zvec
      if with_cnt:
        z16_v[i, pl.ds(0, 16)] = zvec
        e0_v[i, pl.ds(0, 16)] = onevec
      return carry
    lax.fori_loop(0, C, zfill, 0)

    for k in range(RPT // 80):
      pltpu.sync_copy(z_v.at[pl.ds(0, 80)],
                      acc.at[pl.ds(s * RPT + k * 80, 80)])
      if with_cnt:
        pltpu.sync_copy(z16_v.at[pl.ds(0, 80)],
                        cacc.at[pl.ds(s * RPT + k * 80, 80)])
    plsc.subcore_barrier()

    pltpu.sync_copy(srcr.at[tile], src_v)
    pltpu.sync_copy(dstr.at[tile], dst_v)
    pltpu.sync_copy(wr.at[tile], w_v)

    def wait_rows(sem):
      pltpu.make_async_copy(hl.at[pl.ds(0, C)], z_v, sem).wait()

    for p in range(PREF):
      pltpu.async_copy(hl.at[src_v.at[p]], rows_v.at[p], gsems[p])

    def rnd(jj, carry):
      for b in range(NBUF):
        j = jj * NBUF + b
        nb = (b + PREF) % NBUF
        nj = j + PREF

        @pl.when(jnp.logical_and(nj >= NBUF, nj < CPT))
        def _():
          wait_rows(ssems[nb])

        @pl.when(nj < CPT)
        def _():
          pltpu.async_copy(hl.at[src_v.at[nj]], rows_v.at[nb], gsems[nb])

        wait_rows(gsems[b])

        @plsc.parallel_loop(0, C // 16, unroll=2)
        def _(g):
          wvec = w_v[j, pl.ds(g * 16, 16)]
          for l in range(16):
            wi = jnp.broadcast_to(wvec[l], (16,))
            i = g * 16 + l
            for k in range(nblk):
              rows_v[b, i, pl.ds(k * 16, 16)] = (
                  rows_v[b, i, pl.ds(k * 16, 16)] * wi)

        pltpu.async_copy(rows_v.at[b], acc.at[dst_v.at[j]], ssems[b],
                         add=True)
        if with_cnt:
          pltpu.async_copy(e0_v, cacc.at[dst_v.at[j]], csem, add=True)
      return carry
    lax.fori_loop(0, CPT // NBUF, rnd, 0)

    for b in range(NBUF):
      wait_rows(ssems[b])
    if with_cnt:
      def cdrain(i, carry):
        pltpu.make_async_copy(cnt_o.at[pl.ds(0, C), pl.ds(0, 16)],
                              z16_v, csem).wait()
        return carry
      lax.fori_loop(0, CPT, cdrain, 0)

    plsc.subcore_barrier()
    pltpu.sync_copy(acc.at[pl.ds(s * RPT, RPT)],
                    agg_o.at[pl.ds(s * RPT, RPT), pl.ds(c * dout, dout)])
    if with_cnt:
      pltpu.sync_copy(cacc.at[pl.ds(s * RPT, RPT)],
                      cnt_o.at[pl.ds(s * RPT, RPT), pl.ds(c * 16, 16)])

  return pl.kernel(body, out_type=tuple(out_type), mesh=mesh,
                   scratch_types=scratch,
                   compiler_params=pltpu.CompilerParams(
                       use_tc_tiling_on_sc=False))


_R = 5000


def _tc_in(x, Wl, Wr, b):
  din = x.shape[1]
  dout = Wl.shape[1]

  def body(x_ref, wl_ref, wr_ref, b_ref, hl_ref, hrb_ref):
    xv = x_ref[...]
    hl_ref[...] = jnp.dot(xv, wl_ref[...], preferred_element_type=jnp.float32)
    hrb_ref[...] = (jnp.dot(xv, wr_ref[...], preferred_element_type=jnp.float32)
                    + b_ref[...])

  return pl.pallas_call(
      body,
      grid=(N // _R,),
      in_specs=[
          pl.BlockSpec((_R, din), lambda i: (i, 0)),
          pl.BlockSpec((din, dout), lambda i: (0, 0)),
          pl.BlockSpec((din, dout), lambda i: (0, 0)),
          pl.BlockSpec((1, dout), lambda i: (0, 0)),
      ],
      out_specs=[
          pl.BlockSpec((_R, dout), lambda i: (i, 0)),
          pl.BlockSpec((_R, dout), lambda i: (i, 0)),
      ],
      out_shape=[jax.ShapeDtypeStruct((N, dout), jnp.float32)] * 2,
  )(x, Wl, Wr, b.reshape(1, dout))


def _combine(agg, cnt, hrb, dprev):
  a = agg[:, :dprev] + agg[:, dprev:2 * dprev]
  denom = jnp.maximum(cnt[:, :1] + cnt[:, 16:17], 1.0)
  return jnp.maximum(a / denom + hrb, 0.0)


def _tc_mid(agg, cnt, hrb, Wl, Wr, b):
  dprev = hrb.shape[1]
  dout = Wl.shape[1]

  def body(agg_ref, cnt_ref, hrb_ref, wl, wr, b_ref, hl_o, hrb_o):
    h = _combine(agg_ref[...], cnt_ref[...], hrb_ref[...], dprev)
    hl_o[...] = jnp.dot(h, wl[...], preferred_element_type=jnp.float32)
    hrb_o[...] = (jnp.dot(h, wr[...], preferred_element_type=jnp.float32)
                  + b_ref[...])

  return pl.pallas_call(
      body,
      grid=(N // _R,),
      in_specs=[
          pl.BlockSpec((_R, 128), lambda i: (i, 0)),
          pl.BlockSpec((_R, 128), lambda i: (i, 0)),
          pl.BlockSpec((_R, dprev), lambda i: (i, 0)),
          pl.BlockSpec((dprev, dout), lambda i: (0, 0)),
          pl.BlockSpec((dprev, dout), lambda i: (0, 0)),
          pl.BlockSpec((1, dout), lambda i: (0, 0)),
      ],
      out_specs=[
          pl.BlockSpec((_R, dout), lambda i: (i, 0)),
          pl.BlockSpec((_R, dout), lambda i: (i, 0)),
      ],
      out_shape=[jax.ShapeDtypeStruct((N, dout), jnp.float32)] * 2,
  )(agg, cnt, hrb, Wl, Wr, b.reshape(1, dout))


def _tc_fin(agg, cnt, hrb, Wc, bc):
  dprev = hrb.shape[1]

  def body(agg_ref, cnt_ref, hrb_ref, wc, bc_ref, out):
    h = _combine(agg_ref[...], cnt_ref[...], hrb_ref[...], dprev)
    out[...] = (jnp.dot(h, wc[...], preferred_element_type=jnp.float32)
                + bc_ref[...])

  return pl.pallas_call(
      body,
      grid=(N // _R,),
      in_specs=[
          pl.BlockSpec((_R, 128), lambda i: (i, 0)),
          pl.BlockSpec((_R, 128), lambda i: (i, 0)),
          pl.BlockSpec((_R, dprev), lambda i: (i, 0)),
          pl.BlockSpec((dprev, 1), lambda i: (0, 0)),
          pl.BlockSpec((1, 1), lambda i: (0, 0)),
      ],
      out_specs=pl.BlockSpec((_R, 1), lambda i: (i, 0)),
      out_shape=jax.ShapeDtypeStruct((N, 1), jnp.float32),
  )(agg, cnt, hrb, Wc, bc.reshape(1, 1))


def kernel(x, edge_index, edge_weight, Wl1, Wr1, b1, Wl2, Wr2, b2,
           Wl3, Wr3, b3, Wc, bc):
  src = edge_index[0].reshape(TILES, CPT, C)
  dst = edge_index[1].reshape(TILES, CPT, C)
  w = edge_weight.reshape(TILES, CPT, C)

  hl1, hrb1 = _tc_in(x, Wl1, Wr1, b1)
  agg1, cnt = _sc_agg(64, True)(hl1, src, dst, w)
  hl2, hrb2 = _tc_mid(agg1, cnt, hrb1, Wl2, Wr2, b2)
  (agg2,) = _sc_agg(32, False)(hl2, src, dst, w)
  hl3, hrb3 = _tc_mid(agg2, cnt, hrb2, Wl3, Wr3, b3)
  (agg3,) = _sc_agg(16, False)(hl3, src, dst, w)
  return _tc_fin(agg3, cnt, hrb3, Wc, bc)


# speedup vs baseline: 1.0389x; 1.0389x over previous
import jax
import jax.numpy as jnp
from jax import lax
from jax.experimental import pallas as pl
from jax.experimental.pallas import tpu as pltpu
from jax.experimental.pallas import tpu_sc as plsc

N = 10000
E = 320000
N_PAD = 10240
C = 80
ROWS_E = E // C
N_CORES = 2
N_SUB = 16
TILES = N_CORES * N_SUB
CPT = ROWS_E // TILES
RPT = N_PAD // N_SUB


NBUF = 5
PREF = 3


def _sc_agg(dout, with_cnt):
  nblk = dout // 16
  mesh = plsc.VectorSubcoreMesh(core_axis_name="c", subcore_axis_name="s")
  out_type = [jax.ShapeDtypeStruct((N_PAD, 128), jnp.float32)]
  scratch = [
      pltpu.VMEM((CPT, C), jnp.int32),
      pltpu.VMEM((CPT, C), jnp.int32),
      pltpu.VMEM((CPT, C), jnp.float32),
      pltpu.VMEM((NBUF, C, dout), jnp.float32),
      pltpu.VMEM((C, dout), jnp.float32),
      pltpu.VMEM_SHARED((N_PAD, dout), jnp.float32),
  ] + [pltpu.SemaphoreType.DMA] * (2 * NBUF)
  if with_cnt:
    out_type.append(jax.ShapeDtypeStruct((N_PAD, 128), jnp.float32))
    scratch += [
        pltpu.VMEM((C, 16), jnp.float32),
        pltpu.VMEM((C, 16), jnp.float32),
        pltpu.VMEM_SHARED((N_PAD, 16), jnp.float32),
        pltpu.SemaphoreType.DMA,
    ]

  def body(hl, srcr, dstr, wr, *rest):
    if with_cnt:
      agg_o, cnt_o = rest[0], rest[1]
      (src_v, dst_v, w_v, rows_v, z_v, acc) = rest[2:8]
      gsems = rest[8:8 + NBUF]
      ssems = rest[8 + NBUF:8 + 2 * NBUF]
      e0_v, z16_v, cacc, csem = rest[8 + 2 * NBUF:]
    else:
      agg_o = rest[0]
      (src_v, dst_v, w_v, rows_v, z_v, acc) = rest[1:7]
      gsems = rest[7:7 + NBUF]
      ssems = rest[7 + NBUF:7 + 2 * NBUF]
    c = lax.axis_index("c")
    s = lax.axis_index("s")
    tile = c * N_SUB + s

    zvec = jnp.zeros((16,), jnp.float32)
    onevec = jnp.ones((16,), jnp.float32)

    def zfill(i, carry):
      for k in range(nblk):
        z_v[i, pl.ds(k * 16, 16)] = zvec
      if with_cnt:
        z16_v[i, pl.ds(0, 16)] = zvec
        e0_v[i, pl.ds(0, 16)] = onevec
      return carry
    lax.fori_loop(0, C, zfill, 0)

    for k in range(RPT // 80):
      pltpu.sync_copy(z_v.at[pl.ds(0, 80)],
                      acc.at[pl.ds(s * RPT + k * 80, 80)])
      if with_cnt:
        pltpu.sync_copy(z16_v.at[pl.ds(0, 80)],
                        cacc.at[pl.ds(s * RPT + k * 80, 80)])
    plsc.subcore_barrier()

    pltpu.sync_copy(srcr.at[tile], src_v)
    pltpu.sync_copy(dstr.at[tile], dst_v)
    pltpu.sync_copy(wr.at[tile], w_v)

    def wait_rows(sem):
      pltpu.make_async_copy(hl.at[pl.ds(0, C)], z_v, sem).wait()

    for p in range(PREF):
      pltpu.async_copy(hl.at[src_v.at[p]], rows_v.at[p], gsems[p])

    def rnd(jj, carry):
      for b in range(NBUF):
        j = jj * NBUF + b
        nb = (b + PREF) % NBUF
        nj = j + PREF

        @pl.when(jnp.logical_and(nj >= NBUF, nj < CPT))
        def _():
          wait_rows(ssems[nb])

        @pl.when(nj < CPT)
        def _():
          pltpu.async_copy(hl.at[src_v.at[nj]], rows_v.at[nb], gsems[nb])

        wait_rows(gsems[b])

        @plsc.parallel_loop(0, C // 16, unroll=2)
        def _(g):
          wvec = w_v[j, pl.ds(g * 16, 16)]
          for l in range(16):
            wi = jnp.broadcast_to(wvec[l], (16,))
            i = g * 16 + l
            for k in range(nblk):
              rows_v[b, i, pl.ds(k * 16, 16)] = (
                  rows_v[b, i, pl.ds(k * 16, 16)] * wi)

        pltpu.async_copy(rows_v.at[b], acc.at[dst_v.at[j]], ssems[b],
                         add=True)
        if with_cnt:
          pltpu.async_copy(e0_v, cacc.at[dst_v.at[j]], csem, add=True)
      return carry
    lax.fori_loop(0, CPT // NBUF, rnd, 0)

    for b in range(NBUF):
      wait_rows(ssems[b])
    if with_cnt:
      def cdrain(i, carry):
        pltpu.make_async_copy(cnt_o.at[pl.ds(0, C), pl.ds(0, 16)],
                              z16_v, csem).wait()
        return carry
      lax.fori_loop(0, CPT, cdrain, 0)

    plsc.subcore_barrier()
    pltpu.sync_copy(acc.at[pl.ds(s * RPT, RPT)],
                    agg_o.at[pl.ds(s * RPT, RPT), pl.ds(c * dout, dout)])
    if with_cnt:
      pltpu.sync_copy(cacc.at[pl.ds(s * RPT, RPT)],
                      cnt_o.at[pl.ds(s * RPT, RPT), pl.ds(c * 16, 16)])

  return pl.kernel(body, out_type=tuple(out_type), mesh=mesh,
                   scratch_types=scratch,
                   compiler_params=pltpu.CompilerParams(
                       use_tc_tiling_on_sc=False))


_R = 5000


def _tc_in(x, Wl, Wr, b):
  din = x.shape[1]
  dout = Wl.shape[1]

  def body(x_ref, wl_ref, wr_ref, b_ref, hl_ref, hrb_ref):
    xv = x_ref[...]
    hl_ref[...] = jnp.dot(xv, wl_ref[...], preferred_element_type=jnp.float32)
    hrb_ref[...] = (jnp.dot(xv, wr_ref[...], preferred_element_type=jnp.float32)
                    + b_ref[...])

  return pl.pallas_call(
      body,
      grid=(N // _R,),
      in_specs=[
          pl.BlockSpec((_R, din), lambda i: (i, 0)),
          pl.BlockSpec((din, dout), lambda i: (0, 0)),
          pl.BlockSpec((din, dout), lambda i: (0, 0)),
          pl.BlockSpec((1, dout), lambda i: (0, 0)),
      ],
      out_specs=[
          pl.BlockSpec((_R, dout), lambda i: (i, 0)),
          pl.BlockSpec((_R, dout), lambda i: (i, 0)),
      ],
      out_shape=[jax.ShapeDtypeStruct((N, dout), jnp.float32)] * 2,
  )(x, Wl, Wr, b.reshape(1, dout))


def _combine(agg, cnt, hrb, dprev):
  a = agg[:, :dprev] + agg[:, dprev:2 * dprev]
  denom = jnp.maximum(cnt[:, :1] + cnt[:, 16:17], 1.0)
  return jnp.maximum(a / denom + hrb, 0.0)


def _tc_mid(agg, cnt, hrb, Wl, Wr, b):
  dprev = hrb.shape[1]
  dout = Wl.shape[1]

  def body(agg_ref, cnt_ref, hrb_ref, wl, wr, b_ref, hl_o, hrb_o):
    h = _combine(agg_ref[...], cnt_ref[...], hrb_ref[...], dprev)
    hl_o[...] = jnp.dot(h, wl[...], preferred_element_type=jnp.float32)
    hrb_o[...] = (jnp.dot(h, wr[...], preferred_element_type=jnp.float32)
                  + b_ref[...])

  return pl.pallas_call(
      body,
      grid=(N // _R,),
      in_specs=[
          pl.BlockSpec((_R, 128), lambda i: (i, 0)),
          pl.BlockSpec((_R, 128), lambda i: (i, 0)),
          pl.BlockSpec((_R, dprev), lambda i: (i, 0)),
          pl.BlockSpec((dprev, dout), lambda i: (0, 0)),
          pl.BlockSpec((dprev, dout), lambda i: (0, 0)),
          pl.BlockSpec((1, dout), lambda i: (0, 0)),
      ],
      out_specs=[
          pl.BlockSpec((_R, dout), lambda i: (i, 0)),
          pl.BlockSpec((_R, dout), lambda i: (i, 0)),
      ],
      out_shape=[jax.ShapeDtypeStruct((N, dout), jnp.float32)] * 2,
  )(agg, cnt, hrb, Wl, Wr, b.reshape(1, dout))


def _tc_fin(agg, cnt, hrb, Wc, bc):
  dprev = hrb.shape[1]

  def body(agg_ref, cnt_ref, hrb_ref, wc, bc_ref, out):
    h = _combine(agg_ref[...], cnt_ref[...], hrb_ref[...], dprev)
    out[...] = (jnp.dot(h, wc[...], preferred_element_type=jnp.float32)
                + bc_ref[...])

  return pl.pallas_call(
      body,
      grid=(N // _R,),
      in_specs=[
          pl.BlockSpec((_R, 128), lambda i: (i, 0)),
          pl.BlockSpec((_R, 128), lambda i: (i, 0)),
          pl.BlockSpec((_R, dprev), lambda i: (i, 0)),
          pl.BlockSpec((dprev, 1), lambda i: (0, 0)),
          pl.BlockSpec((1, 1), lambda i: (0, 0)),
      ],
      out_specs=pl.BlockSpec((_R, 1), lambda i: (i, 0)),
      out_shape=jax.ShapeDtypeStruct((N, 1), jnp.float32),
  )(agg, cnt, hrb, Wc, bc.reshape(1, 1))


def kernel(x, edge_index, edge_weight, Wl1, Wr1, b1, Wl2, Wr2, b2,
           Wl3, Wr3, b3, Wc, bc):
  src = edge_index[0].reshape(TILES, CPT, C)
  dst = edge_index[1].reshape(TILES, CPT, C)
  w = edge_weight.reshape(TILES, CPT, C)

  hl1, hrb1 = _tc_in(x, Wl1, Wr1, b1)
  agg1, cnt = _sc_agg(64, True)(hl1, src, dst, w)
  hl2, hrb2 = _tc_mid(agg1, cnt, hrb1, Wl2, Wr2, b2)
  (agg2,) = _sc_agg(32, False)(hl2, src, dst, w)
  hl3, hrb3 = _tc_mid(agg2, cnt, hrb2, Wl3, Wr3, b3)
  (agg3,) = _sc_agg(16, False)(hl3, src, dst, w)
  return _tc_fin(agg3, cnt, hrb3, Wc, bc)
